# Initial kernel scaffold; baseline (speedup 1.0000x reference)
import jax
import jax.numpy as jnp
from jax.experimental import pallas as pl


def kernel(x, type_emb, nW1, nb1, nW2, nb2, rW1, rb1, rW2, rb2, temperature):
    raise NotImplementedError("write your pallas kernel here")


# speedup vs baseline: 7.8045x; 7.8045x over previous
import functools

import jax
import jax.numpy as jnp
import numpy as np
from jax import lax
from jax.experimental import pallas as pl
from jax.experimental.pallas import tpu as pltpu

_EXPERT_TYPES = (0, 1, 2, 0, 1, 2, 0, 1)
_E = 8
_TOP_K = 2
_TBLK = 256

_INV_SQRT2 = float(1.0 / np.sqrt(2.0))


def _gelu(t):
    return 0.5 * t * (1.0 + lax.erf(t * _INV_SQRT2))


def _softplus(t):
    return jnp.maximum(t, 0.0) + jnp.log(1.0 + jnp.exp(-jnp.abs(t)))


def _bdot(a, b):
    return jnp.dot(a.astype(jnp.bfloat16), b,
                   preferred_element_type=jnp.float32)


def _prep_body(tf_ref, w1tT_ref, rb1_ref, c_ref):
    c_ref[...] = _bdot(tf_ref[...], w1tT_ref[...]) + rb1_ref[...]


def _main_body(x_ref, w1xT_ref, c_ref, w2T_ref, nW1T_ref, nb1_ref, nW2T_ref,
               nb2_ref, rb2_ref, noise_ref, ct_ref, rout_ref, idx_ref):
    xb = x_ref[...]
    xp = _bdot(xb, w1xT_ref[...])
    cols = []
    for e in range(_E):
        h = _gelu(xp + c_ref[e:e + 1, :])
        le = _bdot(h, w2T_ref[...]) + rb2_ref[...]
        cols.append(jnp.mean(le, axis=1, keepdims=True))
    logits = jnp.concatenate(cols, axis=1)

    nh = _gelu(_bdot(xb, nW1T_ref[...]) + nb1_ref[...])
    nstd = _softplus(_softplus(_bdot(nh, nW2T_ref[...]) + nb2_ref[...]))

    ie = lax.broadcasted_iota(jnp.int32, (1, _E), 1)
    wide = jnp.zeros((1, _E), jnp.bool_)
    for j, ty in enumerate(_EXPERT_TYPES):
        if ty == 1:
            wide = wide | (ie == j)
    wbias = jnp.where(wide, 0.3, 0.0)
    ct = ct_ref[0, 0]
    noisy = logits + ct * (noise_ref[...] * nstd) + wbias

    t = noisy.shape[0]
    iota_e = lax.broadcasted_iota(jnp.int32, (t, _E), 1)
    m1 = jnp.max(noisy, axis=1, keepdims=True)
    i1 = jnp.min(jnp.where(noisy == m1, iota_e, _E), axis=1, keepdims=True)
    masked = jnp.where(iota_e == i1, -jnp.inf, noisy)
    m2 = jnp.max(masked, axis=1, keepdims=True)
    i2 = jnp.min(jnp.where(masked == m2, iota_e, _E), axis=1, keepdims=True)
    sel = (iota_e == i1) | (iota_e == i2)
    p = jnp.where(sel, jnp.exp(noisy - m1), 0.0)
    rout_ref[...] = p / jnp.sum(p, axis=1, keepdims=True)
    idx_ref[...] = jnp.concatenate([i1, i2], axis=1)


def kernel(x, type_emb, nW1, nb1, nW2, nb2, rW1, rb1, rW2, rb2, temperature):
    B, S, D = x.shape
    N = B * S
    F = rW1.shape[0]
    H = nW1.shape[0]
    x2 = x.reshape(N, D).astype(jnp.float32)

    et = jnp.asarray(np.array(_EXPERT_TYPES, dtype=np.int32))
    tf = jnp.take(type_emb, et, axis=0)
    bf16 = jnp.bfloat16
    W1xT = rW1[:, :D].T.astype(bf16)
    W1tT = rW1[:, D:].T.astype(bf16)
    w2T = rW2.T.astype(bf16)
    nW1T = nW1.T.astype(bf16)
    nW2T = nW2.T.astype(bf16)

    c = pl.pallas_call(
        _prep_body,
        out_shape=jax.ShapeDtypeStruct((_E, F), jnp.float32),
    )(tf, W1tT, rb1.reshape(1, F))

    noise = jax.random.normal(jax.random.key(42), (B, S, _E),
                              dtype=jnp.float32).reshape(N, _E)
    ct = jnp.clip(temperature * (0.95 ** (S // 100)), 0.05, 3.0)
    ct = ct.reshape(1, 1).astype(jnp.float32)

    nblk = N // _TBLK
    rout, idx = pl.pallas_call(
        _main_body,
        grid=(nblk,),
        in_specs=[
            pl.BlockSpec((_TBLK, D), lambda i: (i, 0)),
            pl.BlockSpec((D, F), lambda i: (0, 0)),
            pl.BlockSpec((_E, F), lambda i: (0, 0)),
            pl.BlockSpec((F, _E), lambda i: (0, 0)),
            pl.BlockSpec((D, H), lambda i: (0, 0)),
            pl.BlockSpec((1, H), lambda i: (0, 0)),
            pl.BlockSpec((H, _E), lambda i: (0, 0)),
            pl.BlockSpec((1, _E), lambda i: (0, 0)),
            pl.BlockSpec((1, _E), lambda i: (0, 0)),
            pl.BlockSpec((_TBLK, _E), lambda i: (i, 0)),
            pl.BlockSpec(memory_space=pltpu.SMEM),
        ],
        out_specs=[
            pl.BlockSpec((_TBLK, _E), lambda i: (i, 0)),
            pl.BlockSpec((_TBLK, _TOP_K), lambda i: (i, 0)),
        ],
        out_shape=[
            jax.ShapeDtypeStruct((N, _E), jnp.float32),
            jax.ShapeDtypeStruct((N, _TOP_K), jnp.int32),
        ],
    )(x2, W1xT, c, w2T, nW1T, nb1.reshape(1, H), nW2T, nb2.reshape(1, _E),
      rb2.reshape(1, _E), noise, ct)

    return (rout.reshape(B, S, _E).astype(x.dtype), idx.reshape(B, S, _TOP_K))
